# baseline (speedup 1.0000x reference)
import jax
import jax.numpy as jnp
from jax.experimental import pallas as pl

N_NODES = 10000


def _final_mlp_kernel(h_ref, w1_ref, b1_ref, w2_ref, b2_ref, o_ref):
    h = h_ref[...]
    a = h @ w1_ref[...] + b1_ref[...][None, :]
    z = jnp.where(a > 0, a, jnp.exp(jnp.minimum(a, 0.0)) - 1.0)
    o_ref[...] = z @ w2_ref[...] + b2_ref[...][None, :]


def _gatv2(x, src, dst, edge_attr, p, H, C, num_nodes):
    xl = (x @ p["Wl"] + p["bl"]).reshape(num_nodes, H, C)
    xr = (x @ p["Wr"] + p["br"]).reshape(num_nodes, H, C)
    xe = (edge_attr @ p["We"]).reshape(-1, H, C)
    m = xl[src] + xr[dst] + xe
    m = jax.nn.leaky_relu(m, 0.2)
    logits = jnp.sum(m * p["att"][None, :, :], axis=-1)
    lmax = jax.ops.segment_max(logits, dst, num_segments=num_nodes)
    lmax = jnp.where(jnp.isfinite(lmax), lmax, 0.0)
    ex = jnp.exp(logits - lmax[dst])
    denom = jax.ops.segment_sum(ex, dst, num_segments=num_nodes)
    alpha = ex / (denom[dst] + 1e-16)
    msg = alpha[:, :, None] * xl[src]
    out = jax.ops.segment_sum(msg, dst, num_segments=num_nodes)
    return out.reshape(num_nodes, H * C) + p["bias"]


def kernel(x, edge_index, edge_attr, params):
    src = edge_index[0]
    dst = edge_index[1]
    h = jax.nn.elu(x @ params["linA"]["W"] + params["linA"]["b"])
    h = jax.nn.elu(h @ params["linB"]["W"] + params["linB"]["b"])
    h = jax.nn.elu(_gatv2(h, src, dst, edge_attr, params["conv1"], 8, 24, N_NODES))
    h = jax.nn.elu(_gatv2(h, src, dst, edge_attr, params["conv2"], 8, 24, N_NODES))
    h = jax.nn.elu(_gatv2(h, src, dst, edge_attr, params["conv3"], 8, 24, N_NODES))
    h = jax.nn.elu(_gatv2(h, src, dst, edge_attr, params["conv4"], 8, 8, N_NODES))
    out = pl.pallas_call(
        _final_mlp_kernel,
        out_shape=jax.ShapeDtypeStruct((N_NODES, 1), jnp.float32),
    )(h, params["lin1"]["W"], params["lin1"]["b"],
      params["lin2"]["W"], params["lin2"]["b"])
    return out


# speedup vs baseline: 6.1882x; 6.1882x over previous
import functools
import jax
import jax.numpy as jnp
import numpy as np
from jax import lax
from jax.experimental import pallas as pl
from jax.experimental.pallas import tpu as pltpu
from jax.experimental.pallas import tpu_sc as plsc

N_NODES = 10000
N_EDGES = 160000
E_PAD = 163840
CHUNK = 128
N_CHUNKS = E_PAD // (32 * CHUNK)
WACC = 80
DRAIN_STRIDE = 624
DRAIN = 128

_SC_PARAMS = pltpu.CompilerParams(
    use_tc_tiling_on_sc=False, needs_layout_passes=False)
_MESH = plsc.VectorSubcoreMesh(core_axis_name="c", subcore_axis_name="s")


def _elu(x):
    return jnp.where(x > 0, x, jnp.exp(jnp.minimum(x, 0.0)) - 1.0)


def _k0_body(x_ref, wa_ref, ba_ref, wb_ref, bb_ref, wl_ref, bl_ref,
             wr_ref, br_ref, xl_ref, xr_ref):
    h = _elu(x_ref[...] @ wa_ref[...] + ba_ref[...][None, :])
    h = _elu(h @ wb_ref[...] + bb_ref[...][None, :])
    xl_ref[...] = h @ wl_ref[...] + bl_ref[...][None, :]
    xr_ref[...] = h @ wr_ref[...] + br_ref[...][None, :]


def _xe_body(ea_ref, we_ref, xe_ref):
    xe_ref[...] = ea_ref[...] @ we_ref[...]


def _combine_body(o0_ref, o1_ref, d0_ref, d1_ref, p_ref, bias_ref,
                  wl_ref, bl_ref, wr_ref, br_ref, xl_ref, xr_ref):
    d = d0_ref[...] + d1_ref[...]
    r = 1.0 / (d + 1e-16)
    bc = r @ p_ref[...]
    h = _elu((o0_ref[...] + o1_ref[...]) * bc + bias_ref[...][None, :])
    xl_ref[...] = h @ wl_ref[...] + bl_ref[...][None, :]
    xr_ref[...] = h @ wr_ref[...] + br_ref[...][None, :]


def _final_body(o0_ref, o1_ref, d0_ref, d1_ref, p_ref, bias_ref,
                w1_ref, b1_ref, w2_ref, b2_ref, out_ref):
    d = d0_ref[...] + d1_ref[...]
    r = 1.0 / (d + 1e-16)
    bc = r @ p_ref[...]
    h = _elu((o0_ref[...] + o1_ref[...]) * bc + bias_ref[...][None, :])
    t = _elu(h @ w1_ref[...] + b1_ref[...][None, :])
    out_ref[...] = t @ w2_ref[...] + b2_ref[...][None, :]


def _make_sc_ex(C):
    F = 8 * C

    @functools.partial(
        pl.kernel,
        mesh=_MESH,
        compiler_params=_SC_PARAMS,
        out_type=jax.ShapeDtypeStruct((E_PAD, 16), jnp.float32),
        scratch_types=[
            pltpu.VMEM((CHUNK,), jnp.int32),
            pltpu.VMEM((CHUNK,), jnp.int32),
            pltpu.VMEM((CHUNK, F), jnp.float32),
            pltpu.VMEM((CHUNK, F), jnp.float32),
            pltpu.VMEM((CHUNK, F), jnp.float32),
            pltpu.VMEM((CHUNK, 16), jnp.float32),
            pltpu.VMEM((F, 16), jnp.float32),
        ],
    )
    def sc_ex(xl_hbm, xr_hbm, xe_hbm, src_hbm, dst_hbm, att_hbm, ex_hbm,
              sid_v, did_v, xl_v, xr_v, xe_v, ex_v, att_v):
        cid = lax.axis_index("c")
        sid = lax.axis_index("s")
        wid = cid * 16 + sid
        iota = lax.iota(jnp.int32, 16)
        zero16 = jnp.zeros((16,), jnp.float32)

        pltpu.sync_copy(att_hbm, att_v)
        def _zr(r, _):
            ex_v[r, pl.ds(0, 16)] = zero16
            return 0
        lax.fori_loop(0, CHUNK, _zr, 0)

        ebase = wid * (N_CHUNKS * CHUNK)

        def _chunk(ch, _):
            estart = ebase + ch * CHUNK
            pltpu.sync_copy(src_hbm.at[pl.ds(estart, CHUNK)], sid_v)
            pltpu.sync_copy(dst_hbm.at[pl.ds(estart, CHUNK)], did_v)
            pltpu.sync_copy(xe_hbm.at[pl.ds(estart, CHUNK)], xe_v)
            pltpu.sync_copy(xl_hbm.at[sid_v], xl_v)
            pltpu.sync_copy(xr_hbm.at[did_v], xr_v)

            def _group(g, __):
                rowv = g * 16 + iota
                mask = (estart + rowv) < N_EDGES
                for h in range(8):
                    acc = jnp.zeros((16,), jnp.float32)
                    for c in range(C):
                        j = h * C + c
                        colv = jnp.full((16,), j, jnp.int32)
                        a = plsc.load_gather(xl_v, [rowv, colv])
                        b = plsc.load_gather(xr_v, [rowv, colv])
                        e = plsc.load_gather(xe_v, [rowv, colv])
                        m = a + b + e
                        m = jnp.maximum(m, 0.2 * m)
                        acc = acc + m * att_v[j, pl.ds(0, 16)]
                    lg = jnp.minimum(jnp.maximum(acc, -60.0), 60.0)
                    ex = jnp.where(mask, jnp.exp(lg), 0.0)
                    plsc.store_scatter(
                        ex_v, [rowv, jnp.full((16,), h, jnp.int32)], ex)
                return 0

            lax.fori_loop(0, CHUNK // 16, _group, 0)
            pltpu.sync_copy(ex_v, ex_hbm.at[pl.ds(estart, CHUNK)])
            return 0

        lax.fori_loop(0, N_CHUNKS, _chunk, 0)

    return sc_ex


def _make_sc_msg(C, J0):
    F = 8 * C

    @functools.partial(
        pl.kernel,
        mesh=_MESH,
        compiler_params=_SC_PARAMS,
        out_type=jax.ShapeDtypeStruct((2, N_NODES, WACC), jnp.float32),
        scratch_types=[
            pltpu.VMEM((CHUNK,), jnp.int32),
            pltpu.VMEM((CHUNK,), jnp.int32),
            pltpu.VMEM((CHUNK, F), jnp.float32),
            pltpu.VMEM((CHUNK, 16), jnp.float32),
            pltpu.VMEM((CHUNK, WACC), jnp.float32),
            pltpu.VMEM_SHARED((N_NODES, WACC), jnp.float32),
        ],
    )
    def sc_msg(xl_hbm, src_hbm, dst_hbm, ex_hbm, out_hbm,
               sid_v, did_v, xl_v, ex_v, msg_v, acc_sh):
        cid = lax.axis_index("c")
        sid = lax.axis_index("s")
        wid = cid * 16 + sid
        iota = lax.iota(jnp.int32, 16)
        zero16 = jnp.zeros((16,), jnp.float32)

        def _zr(r, _):
            def _zc(c, __):
                msg_v[r, pl.ds(c * 16, 16)] = zero16
                return 0
            return lax.fori_loop(0, WACC // 16, _zc, 0)
        lax.fori_loop(0, CHUNK, _zr, 0)
        for i in range(5):
            pltpu.sync_copy(msg_v.at[pl.ds(0, DRAIN)],
                            acc_sh.at[pl.ds(sid * DRAIN_STRIDE + i * DRAIN,
                                            DRAIN)])
        plsc.subcore_barrier()

        ebase = wid * (N_CHUNKS * CHUNK)

        def _chunk(ch, _):
            estart = ebase + ch * CHUNK
            pltpu.sync_copy(src_hbm.at[pl.ds(estart, CHUNK)], sid_v)
            pltpu.sync_copy(dst_hbm.at[pl.ds(estart, CHUNK)], did_v)
            pltpu.sync_copy(ex_hbm.at[pl.ds(estart, CHUNK)], ex_v)
            pltpu.sync_copy(xl_hbm.at[sid_v], xl_v)

            def _group(g, __):
                rowv = g * 16 + iota
                exs = []
                for h in range(8):
                    hv = jnp.full((16,), h, jnp.int32)
                    ex = plsc.load_gather(ex_v, [rowv, hv])
                    exs.append(ex)
                    plsc.store_scatter(
                        msg_v, [rowv, jnp.full((16,), 64 + h, jnp.int32)], ex)
                for jj in range(64):
                    j = J0 + jj
                    h = j // C
                    colv = jnp.full((16,), j, jnp.int32)
                    a = plsc.load_gather(xl_v, [rowv, colv])
                    plsc.store_scatter(
                        msg_v, [rowv, jnp.full((16,), jj, jnp.int32)],
                        a * exs[h])
                return 0

            lax.fori_loop(0, CHUNK // 16, _group, 0)
            pltpu.sync_copy(msg_v, acc_sh.at[did_v], add=True)
            return 0

        lax.fori_loop(0, N_CHUNKS, _chunk, 0)
        plsc.subcore_barrier()

        for i in range(5):
            r0 = sid * DRAIN_STRIDE + i * DRAIN
            pltpu.sync_copy(acc_sh.at[pl.ds(r0, DRAIN)],
                            msg_v.at[pl.ds(0, DRAIN)])
            pltpu.sync_copy(msg_v.at[pl.ds(0, DRAIN)],
                            out_hbm.at[cid, pl.ds(r0, DRAIN)])

    return sc_msg


_SC_EX_24 = _make_sc_ex(24)
_SC_EX_8 = _make_sc_ex(8)
_SC_MSG_24 = [_make_sc_msg(24, j0) for j0 in (0, 64, 128)]
_SC_MSG_8 = [_make_sc_msg(8, 0)]


def _head_selector(H, C):
    p = np.zeros((16, H * C), np.float32)
    for h in range(H):
        p[h, h * C:(h + 1) * C] = 1.0
    return jnp.asarray(p)


def kernel(x, edge_index, edge_attr, params):
    src = edge_index[0]
    dst = edge_index[1]
    pad = E_PAD - N_EDGES
    src_p = jnp.concatenate([src, jnp.zeros((pad,), jnp.int32)])
    dst_p = jnp.concatenate([dst,
                             (jnp.arange(pad, dtype=jnp.int32) % N_NODES)])
    ea_p = jnp.concatenate(
        [edge_attr, jnp.zeros((pad, edge_attr.shape[1]), jnp.float32)])

    def xe_of(p, F):
        return pl.pallas_call(
            _xe_body,
            grid=(E_PAD // 2048,),
            in_specs=[
                pl.BlockSpec((2048, 7), lambda i: (i, 0)),
                pl.BlockSpec((7, F), lambda i: (0, 0)),
            ],
            out_specs=pl.BlockSpec((2048, F), lambda i: (i, 0)),
            out_shape=jax.ShapeDtypeStruct((E_PAD, F), jnp.float32),
        )(ea_p, p["We"])

    p1, p2, p3, p4 = (params["conv1"], params["conv2"], params["conv3"],
                      params["conv4"])

    def _w(shape):
        return pl.BlockSpec(shape, lambda i: tuple(0 for _ in shape))

    NB = 2000
    xl, xr = pl.pallas_call(
        _k0_body,
        grid=(N_NODES // NB,),
        in_specs=[pl.BlockSpec((NB, 23), lambda i: (i, 0)),
                  _w((23, 23)), _w((23,)), _w((23, 23)), _w((23,)),
                  _w((23, 192)), _w((192,)), _w((23, 192)), _w((192,))],
        out_specs=[pl.BlockSpec((NB, 192), lambda i: (i, 0))] * 2,
        out_shape=[jax.ShapeDtypeStruct((N_NODES, 192), jnp.float32)] * 2,
    )(x, params["linA"]["W"], params["linA"]["b"],
      params["linB"]["W"], params["linB"]["b"],
      p1["Wl"], p1["bl"], p1["Wr"], p1["br"])

    p24 = _head_selector(8, 24)
    p8 = _head_selector(8, 8)

    def run_layer(xl, xr, pconv, F, sc_ex, sc_msgs):
        xe = xe_of(pconv, F)
        att_splat = jnp.repeat(pconv["att"].reshape(F, 1), 16, axis=1)
        ex = sc_ex(xl, xr, xe, src_p, dst_p, att_splat)
        parts = [m(xl, src_p, dst_p, ex) for m in sc_msgs]
        o = jnp.concatenate([p[:, :, :64] for p in parts], axis=2)
        d = parts[0][:, :, 64:]
        return o, d

    def combine_prep(o, d, pconv, psel, pnext):
        F = o.shape[2]
        Fo = pnext["Wl"].shape[1]
        return pl.pallas_call(
            _combine_body,
            grid=(N_NODES // NB,),
            in_specs=[pl.BlockSpec((NB, F), lambda i: (i, 0)),
                      pl.BlockSpec((NB, F), lambda i: (i, 0)),
                      pl.BlockSpec((NB, 16), lambda i: (i, 0)),
                      pl.BlockSpec((NB, 16), lambda i: (i, 0)),
                      _w((16, F)), _w((F,)),
                      _w((F, Fo)), _w((Fo,)), _w((F, Fo)), _w((Fo,))],
            out_specs=[pl.BlockSpec((NB, Fo), lambda i: (i, 0))] * 2,
            out_shape=[jax.ShapeDtypeStruct((N_NODES, Fo), jnp.float32)] * 2,
        )(o[0], o[1], d[0], d[1], psel, pconv["bias"],
          pnext["Wl"], pnext["bl"], pnext["Wr"], pnext["br"])

    o, d = run_layer(xl, xr, p1, 192, _SC_EX_24, _SC_MSG_24)
    xl, xr = combine_prep(o, d, p1, p24, p2)
    o, d = run_layer(xl, xr, p2, 192, _SC_EX_24, _SC_MSG_24)
    xl, xr = combine_prep(o, d, p2, p24, p3)
    o, d = run_layer(xl, xr, p3, 192, _SC_EX_24, _SC_MSG_24)
    xl, xr = combine_prep(o, d, p3, p24, p4)
    o, d = run_layer(xl, xr, p4, 64, _SC_EX_8, _SC_MSG_8)

    out = pl.pallas_call(
        _final_body,
        out_shape=jax.ShapeDtypeStruct((N_NODES, 1), jnp.float32),
    )(o[0], o[1], d[0], d[1], p8, p4["bias"],
      params["lin1"]["W"], params["lin1"]["b"],
      params["lin2"]["W"], params["lin2"]["b"])
    return out
